# Initial kernel scaffold; baseline (speedup 1.0000x reference)
import jax
import jax.numpy as jnp
from jax.experimental import pallas as pl


def kernel(xs, edge_index):
    raise NotImplementedError("write your pallas kernel here")


# speedup vs baseline: 10.4565x; 10.4565x over previous
import functools

import jax
import jax.numpy as jnp
from jax import lax
from jax.experimental import pallas as pl
from jax.experimental.pallas import tpu as pltpu
from jax.experimental.pallas import tpu_sc as plsc

N = 10000
D = 128
DH = D // 2
E = 320000

NC = 2
NS = 16
EPW = E // NS
K = 125
NCHUNK = EPW // K
NP = 10240
RPT = NP // NS
ZR = 128
CW = 16

_mesh = plsc.VectorSubcoreMesh(core_axis_name="c", subcore_axis_name="s")


@functools.partial(
    pl.kernel,
    out_type=[
        jax.ShapeDtypeStruct((NC, NP, DH), jnp.float32),
        jax.ShapeDtypeStruct((NP, CW), jnp.float32),
    ],
    mesh=_mesh,
    compiler_params=pltpu.CompilerParams(use_tc_tiling_on_sc=False),
    scratch_types=[
        pltpu.VMEM((NCHUNK, K), jnp.int32),
        pltpu.VMEM((NCHUNK, K), jnp.int32),
        pltpu.VMEM((K, DH), jnp.float32),
        pltpu.VMEM((K, DH), jnp.float32),
        pltpu.VMEM((ZR, DH), jnp.float32),
        pltpu.VMEM((ZR, CW), jnp.float32),
        pltpu.VMEM((K, CW), jnp.float32),
        pltpu.VMEM_SHARED((NP, DH), jnp.float32),
        pltpu.VMEM_SHARED((NP, CW), jnp.float32),
        pltpu.SemaphoreType.DMA,
        pltpu.SemaphoreType.DMA,
    ],
)
def _sc_accumulate(edges_hbm, xs_lo_hbm, xs_hi_hbm, sums_out, cnts_out,
                   src_v, dst_v, buf_a, buf_b, zero_v, czero_v, ones_v,
                   sums_sh, cnts_sh, sem_a, sem_b):
    cid = lax.axis_index("c")
    sid = lax.axis_index("s")

    pltpu.sync_copy(edges_hbm.at[0, sid], src_v)
    pltpu.sync_copy(edges_hbm.at[1, sid], dst_v)

    def _zero_row(i, _):
        for j in range(DH // 16):
            zero_v[i, pl.ds(j * 16, 16)] = jnp.zeros((16,), jnp.float32)
        czero_v[i] = jnp.zeros((16,), jnp.float32)
        return 0

    lax.fori_loop(0, ZR, _zero_row, 0)

    def _one_row(i, _):
        ones_v[i] = jnp.full((16,), 1.0, jnp.float32)
        return 0

    lax.fori_loop(0, K, _one_row, 0)

    for t in range(RPT // ZR):
        row = sid * RPT + t * ZR
        pltpu.sync_copy(zero_v, sums_sh.at[pl.ds(row, ZR)])
        pltpu.sync_copy(czero_v, cnts_sh.at[pl.ds(row, ZR)])
    plsc.subcore_barrier()

    def _main_loop(xs_ref, do_counts):
        def _gather(j, buf, sem):
            return pltpu.async_copy(xs_ref.at[src_v.at[j]], buf, sem)

        def _scatter(j, buf):
            pltpu.sync_copy(buf, sums_sh.at[dst_v.at[j]], add=True)
            if do_counts:
                pltpu.sync_copy(ones_v, cnts_sh.at[dst_v.at[j]], add=True)

        _gather(0, buf_a, sem_a).wait()

        def _body(p, _):
            j = 2 * p
            _gather(j + 1, buf_b, sem_b)
            _scatter(j, buf_a)

            @pl.when(j + 2 < NCHUNK)
            def _():
                _gather(j + 2, buf_a, sem_a)

            pltpu.make_async_copy(xs_ref.at[src_v.at[j + 1]], buf_b,
                                  sem_b).wait()
            _scatter(j + 1, buf_b)

            @pl.when(j + 2 < NCHUNK)
            def _():
                pltpu.make_async_copy(xs_ref.at[src_v.at[j + 2]], buf_a,
                                      sem_a).wait()

            return 0

        lax.fori_loop(0, NCHUNK // 2, _body, 0)

    @pl.when(cid == 0)
    def _():
        _main_loop(xs_lo_hbm, do_counts=True)

    @pl.when(cid == 1)
    def _():
        _main_loop(xs_hi_hbm, do_counts=False)

    plsc.subcore_barrier()
    for t in range(RPT // ZR):
        row = sid * RPT + t * ZR
        pltpu.sync_copy(sums_sh.at[pl.ds(row, ZR)],
                        sums_out.at[cid, pl.ds(row, ZR)])

        @pl.when(cid == 0)
        def _():
            pltpu.sync_copy(cnts_sh.at[pl.ds(row, ZR)],
                            cnts_out.at[pl.ds(row, ZR)])


def _merge_body(s0, s1, c, out):
    cnt = jnp.maximum(c[:, 0:1], 1.0)
    out[...] = jnp.concatenate([s0[...], s1[...]], axis=1) / cnt


_ROWS_BLK = 400


def _merge(sums, cnts):
    grid = (N // _ROWS_BLK,)
    s_spec = pl.BlockSpec((_ROWS_BLK, DH), lambda i: (i, 0))
    c_spec = pl.BlockSpec((_ROWS_BLK, CW), lambda i: (i, 0))
    o_spec = pl.BlockSpec((_ROWS_BLK, D), lambda i: (i, 0))
    return pl.pallas_call(
        _merge_body,
        grid=grid,
        in_specs=[s_spec, s_spec, c_spec],
        out_specs=o_spec,
        out_shape=jax.ShapeDtypeStruct((N, D), jnp.float32),
    )(sums[0], sums[1], cnts)


@jax.jit
def kernel(xs, edge_index):
    edges = edge_index.astype(jnp.int32).reshape(2, NS, NCHUNK, K)
    xs2 = xs.reshape(N, 2, DH)
    sums, cnts = _sc_accumulate(edges, xs2[:, 0, :], xs2[:, 1, :])
    return _merge(sums, cnts)
